# Initial kernel scaffold; baseline (speedup 1.0000x reference)
import jax
import jax.numpy as jnp
from jax.experimental import pallas as pl


def kernel(x, edge_index0, cluster1, edge_index1, cluster2, edge_index2, params):
    raise NotImplementedError("write your pallas kernel here")


# speedup vs baseline: 3.1791x; 3.1791x over previous
import functools

import jax
import jax.numpy as jnp
from jax import lax
from jax.experimental import pallas as pl
from jax.experimental.pallas import tpu as pltpu
from jax.experimental.pallas import tpu_sc as plsc

_H = 4
_NC = 2
_NSUB = 16
_NW = _NC * _NSUB
_N0, _N1, _N2 = 10000, 5000, 2500
_NP0, _NP1, _NP2 = 10240, 5120, 2560
_ROWS = 512
_F32 = jnp.float32


def _mm(a, b, prec=lax.Precision.DEFAULT):
    return lax.dot_general(a, b, (((1,), (0,)), ((), ())),
                           precision=prec,
                           preferred_element_type=_F32)


def _prep_from_x(xs, Ws, Us, cvec, copad):
    np_ = xs[0].shape[0]
    yw = 4 * copad
    nx = len(xs)
    grid = (np_ // _ROWS,)
    msk = jnp.concatenate([jnp.ones((4,), _F32),
                           jnp.zeros((12,), _F32)]).reshape(1, 16)

    def body(*refs):
        x_refs = refs[:nx]
        w_refs = refs[nx:2 * nx]
        u_refs = refs[2 * nx:3 * nx]
        c_ref, mk_ref = refs[3 * nx], refs[3 * nx + 1]
        y_ref, ta_ref, tb_ref = refs[3 * nx + 2:3 * nx + 5]
        y = _mm(x_refs[0][...], w_refs[0][...])
        t = _mm(x_refs[0][...], u_refs[0][...], lax.Precision.HIGHEST)
        for i in range(1, nx):
            y = y + _mm(x_refs[i][...], w_refs[i][...])
            t = t + _mm(x_refs[i][...], u_refs[i][...], lax.Precision.HIGHEST)
        y_ref[...] = y
        ta_ref[...] = t * mk_ref[...] + c_ref[...]
        tb_ref[...] = -t * mk_ref[...]

    in_specs = (
        [pl.BlockSpec((_ROWS, int(x.shape[1])), lambda i: (i, 0)) for x in xs]
        + [pl.BlockSpec(tuple(w.shape), lambda i: (0, 0)) for w in Ws]
        + [pl.BlockSpec(tuple(u.shape), lambda i: (0, 0)) for u in Us]
        + [pl.BlockSpec((1, 16), lambda i: (0, 0))] * 2
    )
    out_specs = [pl.BlockSpec((_ROWS, yw), lambda i: (i, 0)),
                 pl.BlockSpec((_ROWS, 16), lambda i: (i, 0)),
                 pl.BlockSpec((_ROWS, 16), lambda i: (i, 0))]
    return pl.pallas_call(
        body, grid=grid, in_specs=in_specs, out_specs=out_specs,
        out_shape=[jax.ShapeDtypeStruct((np_, yw), _F32),
                   jax.ShapeDtypeStruct((np_, 16), _F32),
                   jax.ShapeDtypeStruct((np_, 16), _F32)],
    )(*xs, *Ws, *Us, cvec, msk)


def _prep_from_pool(S, W, U, cvec):
    np_ = S.shape[1]
    grid = (np_ // _ROWS,)

    msk = jnp.concatenate([jnp.ones((4,), _F32),
                           jnp.zeros((12,), _F32)]).reshape(1, 16)

    def body(s_ref, w_ref, u_ref, c_ref, mk_ref, y_ref, ta_ref, tb_ref,
             x_ref):
        s2 = s_ref[0] + s_ref[1]
        cnt = jnp.maximum(s2[:, 140:141], 1.0)
        xm = s2[:, :140] / cnt
        y_ref[...] = _mm(xm, w_ref[...])
        t = _mm(xm, u_ref[...], lax.Precision.HIGHEST)
        ta_ref[...] = t * mk_ref[...] + c_ref[...]
        tb_ref[...] = -t * mk_ref[...]
        x_ref[...] = jnp.concatenate(
            [xm, jnp.ones((_ROWS, 1), _F32), jnp.zeros((_ROWS, 3), _F32)],
            axis=1)

    in_specs = [pl.BlockSpec((2, _ROWS, 144), lambda i: (0, i, 0)),
                pl.BlockSpec(tuple(W.shape), lambda i: (0, 0)),
                pl.BlockSpec(tuple(U.shape), lambda i: (0, 0)),
                pl.BlockSpec((1, 16), lambda i: (0, 0)),
                pl.BlockSpec((1, 16), lambda i: (0, 0))]
    out_specs = [pl.BlockSpec((_ROWS, 576), lambda i: (i, 0)),
                 pl.BlockSpec((_ROWS, 16), lambda i: (i, 0)),
                 pl.BlockSpec((_ROWS, 16), lambda i: (i, 0)),
                 pl.BlockSpec((_ROWS, 144), lambda i: (i, 0))]
    return pl.pallas_call(
        body, grid=grid, in_specs=in_specs, out_specs=out_specs,
        out_shape=[jax.ShapeDtypeStruct((np_, 576), _F32),
                   jax.ShapeDtypeStruct((np_, 16), _F32),
                   jax.ShapeDtypeStruct((np_, 16), _F32),
                   jax.ShapeDtypeStruct((np_, 144), _F32)],
    )(S, W, U, cvec, msk)


def _node_update(S, xs, Wres, p):
    np_ = S.shape[1]
    nx = len(xs)
    nw = 0 if Wres is None else len(Wres)
    b = p['b'].reshape(1, 140)
    g = p['g'].reshape(1, 140)
    be = p['beta'].reshape(1, 140)
    grid = (np_ // _ROWS,)

    def body(*refs):
        s_ref = refs[0]
        x_refs = refs[1:1 + nx]
        w_refs = refs[1 + nx:1 + nx + nw]
        b_ref, g_ref, be_ref = refs[1 + nx + nw:4 + nx + nw]
        o_ref = refs[4 + nx + nw]
        s2 = s_ref[0] + s_ref[1]
        cnt = jnp.maximum(s2[:, 140:141], 1.0)
        h = s2[:, :140] / cnt + b_ref[...]
        mu = jnp.mean(h, axis=1, keepdims=True)
        var = jnp.mean((h - mu) ** 2, axis=1, keepdims=True)
        h = (h - mu) * lax.rsqrt(var + 1e-5) * g_ref[...] + be_ref[...]
        h = jnp.maximum(h, 0.0)
        if nw:
            r = _mm(x_refs[0][...], w_refs[0][...])
            for i in range(1, nw):
                r = r + _mm(x_refs[i][...], w_refs[i][...])
        else:
            r = x_refs[0][:, :140]
        h = h + r
        o_ref[...] = jnp.concatenate(
            [h, jnp.ones((_ROWS, 1), _F32), jnp.zeros((_ROWS, 3), _F32)],
            axis=1)

    in_specs = ([pl.BlockSpec((2, _ROWS, 144), lambda i: (0, i, 0))]
                + [pl.BlockSpec((_ROWS, int(x.shape[1])), lambda i: (i, 0))
                   for x in xs]
                + [pl.BlockSpec(tuple(w.shape), lambda i: (0, 0))
                   for w in (Wres or [])]
                + [pl.BlockSpec((1, 140), lambda i: (0, 0))] * 3)
    return pl.pallas_call(
        body, grid=grid, in_specs=in_specs,
        out_specs=pl.BlockSpec((_ROWS, 144), lambda i: (i, 0)),
        out_shape=jax.ShapeDtypeStruct((np_, 144), _F32),
    )(S, *xs, *(Wres or []), b, g, be)


def _final_update(S, x, Wres, b):
    np_ = S.shape[1]
    grid = (np_ // _ROWS,)

    def body(s_ref, x_ref, w_ref, b_ref, o_ref):
        s2 = s_ref[0] + s_ref[1]
        cnt = jnp.maximum(s2[:, 1:2], 1.0)
        h = s2[:, 0:1] / cnt
        r = _mm(x_ref[...], w_ref[...])
        o_ref[...] = h + b_ref[...] + r

    in_specs = [pl.BlockSpec((2, _ROWS, 16), lambda i: (0, i, 0)),
                pl.BlockSpec((_ROWS, 144), lambda i: (i, 0)),
                pl.BlockSpec((144, 8), lambda i: (0, 0)),
                pl.BlockSpec((1, 8), lambda i: (0, 0))]
    return pl.pallas_call(
        body, grid=grid, in_specs=in_specs,
        out_specs=pl.BlockSpec((_ROWS, 8), lambda i: (i, 0)),
        out_shape=jax.ShapeDtypeStruct((np_, 8), _F32),
    )(S, x, Wres, b)


@functools.cache
def _mesh():
    return plsc.VectorSubcoreMesh(core_axis_name="c", subcore_axis_name="s",
                                  num_cores=_NC, num_subcores=_NSUB)


def _edge_sc(src, dst, Y, Ta, Tb, copad, np_, cout, ch):
    ep = src.shape[0] - ch
    epw = ep // _NW
    nch = epw // ch
    assert nch % 2 == 0
    yw = 4 * copad
    nslab = copad // 16
    cslab, clane = cout // 16, cout % 16
    rps = np_ // _NSUB
    nz = rps // ch

    @functools.partial(
        pl.kernel,
        out_type=jax.ShapeDtypeStruct((2 * np_, copad), _F32),
        mesh=_mesh(),
        compiler_params=pltpu.CompilerParams(use_tc_tiling_on_sc=False),
        scratch_types=[
            pltpu.VMEM((epw + ch,), jnp.int32),
            pltpu.VMEM((epw + ch,), jnp.int32),
            [pltpu.VMEM((ch, yw), _F32)] * 2,
            [pltpu.VMEM((ch, 16), _F32)] * 2,
            [pltpu.VMEM((ch, 16), _F32)] * 2,
            [pltpu.VMEM((ch, copad), _F32)] * 2,
            [pltpu.VMEM((ch,), jnp.int32)] * 2,
            pltpu.VMEM_SHARED((np_, copad), _F32),
            [pltpu.SemaphoreType.DMA] * 2,
            [pltpu.SemaphoreType.DMA] * 2,
            [pltpu.SemaphoreType.DMA] * 2,
            [pltpu.SemaphoreType.DMA] * 2,
        ],
    )
    def k(src_h, dst_h, y_h, ta_h, tb_h, s_h,
          isrc, idst, ytb, tsb, tdb, mb, sdst, acc,
          sem_y, sem_t, sem_d, sem_s):
        cid = lax.axis_index("c")
        sid = lax.axis_index("s")
        wid = sid * _NC + cid
        iota16 = lax.iota(jnp.int32, 16)
        zvec = jnp.zeros((16,), _F32)
        lanec = jnp.where(iota16 == clane, 1.0, 0.0).astype(_F32)

        def zrow(r, carry):
            for kk in range(nslab):
                mb[0][r, pl.ds(kk * 16, 16)] = zvec
            return carry
        lax.fori_loop(0, ch, zrow, 0)

        def zcp(j, carry):
            pltpu.sync_copy(mb[0], acc.at[pl.ds(sid * rps + j * ch, ch)])
            return carry
        lax.fori_loop(0, nz, zcp, 0)
        plsc.subcore_barrier()

        base = wid * epw
        pltpu.sync_copy(src_h.at[pl.ds(base, epw + ch)], isrc)
        pltpu.sync_copy(dst_h.at[pl.ds(base, epw + ch)], idst)

        def issue(k_, p):
            pltpu.async_copy(y_h.at[isrc.at[pl.ds(k_ * ch, ch)]],
                             ytb[p], sem_y[p])
            pltpu.async_copy(ta_h.at[isrc.at[pl.ds(k_ * ch, ch)]],
                             tsb[p], sem_t[p])
            pltpu.async_copy(tb_h.at[idst.at[pl.ds(k_ * ch, ch)]],
                             tdb[p], sem_d[p])

        def wait_gather(p):
            pltpu.make_async_copy(y_h.at[isrc.at[pl.ds(0, ch)]],
                                  ytb[p], sem_y[p]).wait()
            pltpu.make_async_copy(ta_h.at[isrc.at[pl.ds(0, ch)]],
                                  tsb[p], sem_t[p]).wait()
            pltpu.make_async_copy(tb_h.at[idst.at[pl.ds(0, ch)]],
                                  tdb[p], sem_d[p]).wait()

        def wait_scatter(p):
            pltpu.make_async_copy(mb[p], acc.at[sdst[p]], sem_s[p]).wait()

        def combine(cc, p):
            def comb(e, carry2):
                ev = jnp.exp(tsb[p][e, :] + tdb[p][e, :])
                q0 = ev[0]
                q1 = ev[1]
                q2 = ev[2]
                q3 = ev[3]
                inv = 1.0 / jnp.full((16,), q0 + q1 + q2 + q3, _F32)
                for ksl in range(nslab):
                    off = ksl * 16
                    v = (q0 * ytb[p][e, pl.ds(off, 16)]
                         + q1 * ytb[p][e, pl.ds(copad + off, 16)]
                         + q2 * ytb[p][e, pl.ds(2 * copad + off, 16)]
                         + q3 * ytb[p][e, pl.ds(3 * copad + off, 16)])
                    v = v * inv
                    if ksl == cslab:
                        v = v + lanec
                    mb[p][e, pl.ds(off, 16)] = v
                return carry2
            lax.fori_loop(0, ch, comb, 0)
            for kk in range(ch // 16):
                sdst[p][pl.ds(kk * 16, 16)] = idst[pl.ds(cc * ch + kk * 16,
                                                         16)]

        issue(0, 0)

        def chunk_pair(i, carry):
            for b in range(2):
                cc = 2 * i + b
                issue(cc + 1, 1 - b)

                @pl.when(cc >= 2)
                def _():
                    wait_scatter(b)
                wait_gather(b)
                combine(cc, b)
                pltpu.async_copy(mb[b], acc.at[sdst[b]], sem_s[b], add=True)
            return carry
        lax.fori_loop(0, nch // 2, chunk_pair, 0)
        wait_gather(0)
        wait_scatter(0)
        wait_scatter(1)

        plsc.subcore_barrier()
        pltpu.sync_copy(acc.at[pl.ds(sid * rps, rps)],
                        s_h.at[pl.ds(cid * np_ + sid * rps, rps)])

    return k(src, dst, Y, Ta, Tb).reshape(2, np_, copad)


def _pool_sc(X, cluster, npb):
    npa = X.shape[0]
    ch = 32
    rpw = npa // _NW
    nch = rpw // ch
    rps = npb // _NSUB
    nz = rps // 32

    @functools.partial(
        pl.kernel,
        out_type=jax.ShapeDtypeStruct((2 * npb, 144), _F32),
        mesh=_mesh(),
        compiler_params=pltpu.CompilerParams(use_tc_tiling_on_sc=False),
        scratch_types=[
            pltpu.VMEM((ch,), jnp.int32),
            pltpu.VMEM((ch, 144), _F32),
            pltpu.VMEM_SHARED((npb, 144), _F32),
        ],
    )
    def k(x_h, cl_h, s_h, idxb, xb, acc):
        cid = lax.axis_index("c")
        sid = lax.axis_index("s")
        wid = sid * _NC + cid
        zvec = jnp.zeros((16,), _F32)

        def zrow(r, carry):
            for kk in range(9):
                xb[r, pl.ds(kk * 16, 16)] = zvec
            return carry
        lax.fori_loop(0, ch, zrow, 0)

        def zcp(j, carry):
            pltpu.sync_copy(xb, acc.at[pl.ds(sid * rps + j * ch, ch)])
            return carry
        lax.fori_loop(0, nz, zcp, 0)
        plsc.subcore_barrier()

        base = wid * rpw

        def chunk(cc, carry):
            r0 = pl.multiple_of(base + cc * ch, ch)
            pltpu.sync_copy(x_h.at[pl.ds(r0, ch)], xb)
            pltpu.sync_copy(cl_h.at[pl.ds(r0, ch)], idxb)
            pltpu.sync_copy(xb, acc.at[idxb], add=True)
            return carry
        lax.fori_loop(0, nch, chunk, 0)
        plsc.subcore_barrier()
        pltpu.sync_copy(acc.at[pl.ds(sid * rps, rps)],
                        s_h.at[pl.ds(cid * npb + sid * rps, rps)])

    return k(X, cluster).reshape(2, npb, 144)


def _unpool_sc(X2, cluster, npa):
    ch = 32
    rpw = npa // _NW
    nch = rpw // ch

    @functools.partial(
        pl.kernel,
        out_type=jax.ShapeDtypeStruct((npa, 144), _F32),
        mesh=_mesh(),
        compiler_params=pltpu.CompilerParams(use_tc_tiling_on_sc=False),
        scratch_types=[
            pltpu.VMEM((ch,), jnp.int32),
            pltpu.VMEM((ch, 144), _F32),
            pltpu.SemaphoreType.DMA,
        ],
    )
    def k(x_h, cl_h, o_h, idxb, xb, sem):
        cid = lax.axis_index("c")
        sid = lax.axis_index("s")
        wid = sid * _NC + cid
        base = wid * rpw

        def chunk(cc, carry):
            r0 = pl.multiple_of(base + cc * ch, ch)
            pltpu.sync_copy(cl_h.at[pl.ds(r0, ch)], idxb)
            pltpu.async_copy(x_h.at[idxb], xb, sem).wait()
            pltpu.sync_copy(xb, o_h.at[pl.ds(r0, ch)])
            return carry
        lax.fori_loop(0, nch, chunk, 0)

    return k(X2, cluster)


def _pad_w_heads(W, cout, copad):
    cin = W.shape[0]
    Wr = W.reshape(cin, _H, cout)
    Wr = jnp.pad(Wr, ((0, 0), (0, 0), (0, copad - cout)))
    return Wr.reshape(cin, _H * copad)


def _pad_rows(M, rp):
    return jnp.pad(M, ((0, rp - M.shape[0]), (0, 0)))


def _u2(u, c):
    cin = u.shape[0]
    U = jnp.concatenate([u, u, jnp.zeros((cin, 8), _F32)], axis=1)
    cv = jnp.pad(c, (0, 12)).reshape(1, 16)
    return U, cv


def _wpk(p, rp, copad=144):
    cout = p['b'].shape[0]
    W = _pad_rows(_pad_w_heads(p['W'], cout, copad), rp)
    U, cv = _u2(p['u'], p['c'])
    return W, _pad_rows(U, rp), cv


def _ceil_to(x, m):
    return ((x + m - 1) // m) * m


def kernel(x, edge_index0, cluster1, edge_index1, cluster2, edge_index2,
           params):
    P = params

    def pad_edges(ei, n_dummy, ch):
        e = ei.shape[1]
        epad = _ceil_to(e, _NW * ch * 2) + ch
        src = jnp.pad(ei[0], (0, epad - e))
        dst = jnp.pad(ei[1], (0, epad - e), constant_values=n_dummy)
        return src, dst

    src0, dst0 = pad_edges(edge_index0, _N0, 16)
    src1, dst1 = pad_edges(edge_index1, _N1, 32)
    src2, dst2 = pad_edges(edge_index2, _N2, 32)
    cl1p = jnp.pad(cluster1, (0, _NP0 - _N0), constant_values=_N1)
    cl2p = jnp.pad(cluster2, (0, _NP1 - _N1), constant_values=_N2)
    x0p = jnp.pad(x, ((0, _NP0 - _N0), (0, 4)))

    def std_block(X, p, src, dst, np_, ch):
        W, U, cv = _wpk(p, 144)
        Y, Ta, Tb = _prep_from_x([X], [W], [U], cv, 144)
        S = _edge_sc(src, dst, Y, Ta, Tb, 144, np_, 140, ch)
        return _node_update(S, [X], None, p)

    def cat_block(Xa, Xb, p, src, dst, np_, ch):
        W6 = _pad_w_heads(p['W'], 140, 144)
        U6, cv = _u2(p['u'], p['c'])
        Ws = [_pad_rows(W6[:140], 144), _pad_rows(W6[140:], 144)]
        Us = [_pad_rows(U6[:140], 144), _pad_rows(U6[140:], 144)]
        Y, Ta, Tb = _prep_from_x([Xa, Xb], Ws, Us, cv, 144)
        S = _edge_sc(src, dst, Y, Ta, Tb, 144, np_, 140, ch)
        Wr = p['Wres']
        Wrs = [_pad_rows(Wr[:140], 144), _pad_rows(Wr[140:], 144)]
        return _node_update(S, [Xa, Xb], Wrs, p)

    p = P[0]
    W, U, cv = _wpk(p, 8)
    Y, Ta, Tb = _prep_from_x([x0p], [W], [U], cv, 144)
    S = _edge_sc(src0, dst0, Y, Ta, Tb, 144, _NP0, 140, 16)
    X = _node_update(S, [x0p], [_pad_rows(p['Wres'], 8)], p)
    X = std_block(X, P[1], src0, dst0, _NP0, 16)
    copy0 = X
    Sp = _pool_sc(X, cl1p, _NP1)
    W2 = _pad_w_heads(P[2]['W'], 140, 144)
    U2v, cv2 = _u2(P[2]['u'], P[2]['c'])
    Y, Ta, Tb, X1 = _prep_from_pool(Sp, W2, U2v, cv2)
    S = _edge_sc(src1, dst1, Y, Ta, Tb, 144, _NP1, 140, 32)
    X1 = _node_update(S, [X1], None, P[2])
    X1 = std_block(X1, P[3], src1, dst1, _NP1, 32)
    copy1 = X1
    Sp2 = _pool_sc(X1, cl2p, _NP2)
    W4 = _pad_w_heads(P[4]['W'], 140, 144)
    U4v, cv4 = _u2(P[4]['u'], P[4]['c'])
    Y, Ta, Tb, X2 = _prep_from_pool(Sp2, W4, U4v, cv4)
    S = _edge_sc(src2, dst2, Y, Ta, Tb, 144, _NP2, 140, 32)
    X2 = _node_update(S, [X2], None, P[4])
    X2 = std_block(X2, P[5], src2, dst2, _NP2, 32)
    xup1 = _unpool_sc(X2, cl2p, _NP1)
    X1 = cat_block(xup1, copy1, P[6], src1, dst1, _NP1, 32)
    X1 = std_block(X1, P[7], src1, dst1, _NP1, 32)
    X1 = std_block(X1, P[8], src1, dst1, _NP1, 32)
    X1 = std_block(X1, P[9], src1, dst1, _NP1, 32)
    xup0 = _unpool_sc(X1, cl1p, _NP0)
    X = cat_block(xup0, copy0, P[10], src0, dst0, _NP0, 16)
    X = std_block(X, P[11], src0, dst0, _NP0, 16)
    X = std_block(X, P[12], src0, dst0, _NP0, 16)
    p = P[13]
    W13 = _pad_rows(_pad_w_heads(p['W'], 1, 16), 144)
    U13, cv13 = _u2(p['u'], p['c'])
    Y, Ta, Tb = _prep_from_x([X], [W13], [_pad_rows(U13, 144)], cv13, 16)
    S = _edge_sc(src0, dst0, Y, Ta, Tb, 16, _NP0, 1, 16)
    Wr13 = jnp.pad(p['Wres'], ((0, 4), (0, 7)))
    b13 = jnp.pad(p['b'], (0, 7)).reshape(1, 8)
    Xf = _final_update(S, X, Wr13, b13)
    return Xf[:_N0, :1]
